# baseline (speedup 1.0000x reference)
import functools
import math

import jax
import jax.numpy as jnp
from jax.experimental import pallas as pl

B, C, H, W = 4, 256, 64, 64
A = 9
REL_THR = 300
RB = 16
AP = 16


def _conv_body(x0, x1, x2, wt, wp, bpre, bproj, out):
    i = pl.program_id(1)
    xs = (x0, x1, x2)
    acc = jnp.zeros((RB * W, C), jnp.float32)
    for dy in range(3):
        start = (i * RB + dy) * W
        for dx in range(3):
            blk = xs[dx][0, pl.ds(start, RB * W), :]
            acc += jnp.dot(blk, wt[dy * 3 + dx],
                           preferred_element_type=jnp.float32)
    y = jnp.maximum(acc + bpre[0][None, :], 0.0)
    out[0] = jnp.dot(y, wp[...], preferred_element_type=jnp.float32) + bproj[0][None, :]


def _logits_tc(feat_map, W_pre, b_pre, W_proj, b_proj):
    x = jnp.transpose(feat_map, (0, 2, 3, 1))
    xp = jnp.pad(x, ((0, 0), (1, 1), (1, 1), (0, 0)))
    xs = [xp[:, :, dx:dx + W, :].reshape(B, (H + 2) * W, C) for dx in range(3)]
    wt = jnp.transpose(W_pre, (2, 3, 1, 0)).reshape(9, C, C)
    wp = jnp.pad(W_proj[:, :, 0, 0].T, ((0, 0), (0, AP - A)))
    bproj_p = jnp.pad(b_proj, (0, AP - A))

    grid = (B, H // RB)
    xspec = pl.BlockSpec((1, (H + 2) * W, C), lambda b, i: (b, 0, 0))
    out = pl.pallas_call(
        _conv_body,
        grid=grid,
        in_specs=[
            xspec, xspec, xspec,
            pl.BlockSpec((9, C, C), lambda b, i: (0, 0, 0)),
            pl.BlockSpec((C, AP), lambda b, i: (0, 0)),
            pl.BlockSpec((1, C), lambda b, i: (0, 0)),
            pl.BlockSpec((1, AP), lambda b, i: (0, 0)),
        ],
        out_specs=pl.BlockSpec((1, RB * W, AP), lambda b, i: (b, i, 0)),
        out_shape=jax.ShapeDtypeStruct((B, H * W, AP), jnp.float32),
    )(xs[0], xs[1], xs[2], wt, wp, b_pre[None, :], bproj_p[None, :])
    return out[:, :, :A].reshape(B, H * W * A)


def kernel(feat_map, W_pre, b_pre, W_proj, b_proj):
    sel_logits = _logits_tc(feat_map, W_pre, b_pre, W_proj, b_proj)
    num_anchors = sel_logits.shape[1]
    _, rel_ids = jax.lax.top_k(sel_logits, REL_THR)
    rel_ids = rel_ids + num_anchors * jnp.arange(B, dtype=rel_ids.dtype)[:, None]
    sel_ids = rel_ids.reshape(-1)
    return sel_logits, sel_ids


# speedup vs baseline: 1.8421x; 1.8421x over previous
import jax
import jax.numpy as jnp
from jax import lax
from jax.experimental import pallas as pl
from jax.experimental.pallas import tpu as pltpu
from jax.experimental.pallas import tpu_sc as plsc

B, C, H, W = 4, 256, 64, 64
A = 9
REL_THR = 300
RB = 16
AP = 16

N = H * W * A
TPB = 8
CH = N // TPB
NV = CH // 16
CAP = 512
NOUT = 304


def _conv_body(x0, x1, x2, wt, wp, bpre, bproj, out):
    i = pl.program_id(1)
    xs = (x0, x1, x2)
    acc = jnp.zeros((RB * W, C), jnp.float32)
    for dy in range(3):
        start = (i * RB + dy) * W
        for dx in range(3):
            blk = xs[dx][0, pl.ds(start, RB * W), :]
            acc += jnp.dot(blk, wt[dy * 3 + dx],
                           preferred_element_type=jnp.float32)
    y = jnp.maximum(acc + bpre[0][None, :], 0.0)
    out[0] = jnp.dot(y, wp[...], preferred_element_type=jnp.float32) + bproj[0][None, :]


def _logits_tc(feat_map, W_pre, b_pre, W_proj, b_proj):
    x = jnp.transpose(feat_map, (0, 2, 3, 1))
    xp = jnp.pad(x, ((0, 0), (1, 1), (1, 1), (0, 0)))
    xs = [xp[:, :, dx:dx + W, :].reshape(B, (H + 2) * W, C) for dx in range(3)]
    wt = jnp.transpose(W_pre, (2, 3, 1, 0)).reshape(9, C, C)
    wp = jnp.pad(W_proj[:, :, 0, 0].T, ((0, 0), (0, AP - A)))
    bproj_p = jnp.pad(b_proj, (0, AP - A))

    grid = (B, H // RB)
    xspec = pl.BlockSpec((1, (H + 2) * W, C), lambda b, i: (b, 0, 0))
    out = pl.pallas_call(
        _conv_body,
        grid=grid,
        in_specs=[
            xspec, xspec, xspec,
            pl.BlockSpec((9, C, C), lambda b, i: (0, 0, 0)),
            pl.BlockSpec((C, AP), lambda b, i: (0, 0)),
            pl.BlockSpec((1, C), lambda b, i: (0, 0)),
            pl.BlockSpec((1, AP), lambda b, i: (0, 0)),
        ],
        out_specs=pl.BlockSpec((1, RB * W, AP), lambda b, i: (b, i, 0)),
        out_shape=jax.ShapeDtypeStruct((B, H * W, AP), jnp.float32),
    )(xs[0], xs[1], xs[2], wt, wp, b_pre[None, :], bproj_p[None, :])
    return out[:, :, :A].reshape(B, H * W * A)


def _topk_body(lg_hbm, out_hbm, vals, keys, h16, hist, hist8, scv, candk,
               candg, cntv, allk, allg, cnts, lout, osum8, osum,
               sh_hist, sh_ck, sh_cg, sh_cnt, sh_out):
    i32 = jnp.int32
    c = lax.axis_index("c")
    s = lax.axis_index("s")
    bic = s // TPB
    chunk = s % TPB
    batch = c * 2 + bic
    lane = jnp.arange(16, dtype=i32)
    zero16 = jnp.zeros((16,), i32)
    one16 = jnp.ones((16,), i32)

    base = batch * N + chunk * CH
    pltpu.sync_copy(lg_hbm.at[pl.ds(base, CH)], vals)

    def zero_h16(j, carry):
        h16[pl.ds(j * 16, 16)] = zero16
        return carry
    lax.fori_loop(0, 256, zero_h16, 0)

    lane_off = lane * 256

    def cvt(j, carry):
        v = vals[pl.ds(j * 16, 16)]
        bits = lax.bitcast_convert_type(v, i32)
        key = jnp.where(bits >= 0, bits, bits ^ jnp.int32(0x7FFFFFFF))
        keys[pl.ds(j * 16, 16)] = key
        d0 = (key >> 24) & 255
        plsc.addupdate_scatter(h16, [lane_off + d0], one16)
        return carry
    lax.fori_loop(0, NV, cvt, 0)

    need = jnp.full((16,), REL_THR, i32)
    prefix = jnp.zeros((16,), i32)

    for p in range(4):
        if p > 0:
            sh_hi = 32 - 8 * p
            pmask_bits = (1 << (8 * p)) - 1
            sh_d = 24 - 8 * p

            def scan(j, carry):
                key = keys[pl.ds(j * 16, 16)]
                pm = ((key >> sh_hi) & pmask_bits) == prefix
                d = (key >> sh_d) & 255
                plsc.addupdate_scatter(h16, [lane_off + d], one16, mask=pm)
                return carry
            lax.fori_loop(0, NV, scan, 0)

        def reduce_h(i, carry):
            acc = zero16
            for l in range(16):
                sl = pl.ds(l * 256 + i * 16, 16)
                acc = acc + h16[sl]
                h16[sl] = zero16
            hist[pl.ds(i * 16, 16)] = acc
            return carry
        lax.fori_loop(0, 16, reduce_h, 0)

        hrow = ((bic * 4 + p) * TPB + chunk) * 256
        pltpu.sync_copy(hist, sh_hist.at[pl.ds(hrow, 256)])
        plsc.subcore_barrier()
        pltpu.sync_copy(sh_hist.at[pl.ds((bic * 4 + p) * TPB * 256, TPB * 256)], hist8)

        hvecs = []
        for i in range(16):
            acc = zero16
            for r in range(TPB):
                acc = acc + hist8[pl.ds(r * 256 + i * 16, 16)]
            hvecs.append(acc)
        tail = jnp.zeros((), i32)
        scs = [None] * 16
        for i in range(15, -1, -1):
            ssum = jnp.flip(plsc.cumsum(jnp.flip(hvecs[i], 0)), 0)
            scs[i] = ssum + tail
            tail = tail + jnp.sum(hvecs[i])
        dtot = zero16
        for i in range(16):
            dtot = dtot + plsc.all_reduce_population_count(scs[i] >= need)
            scv[pl.ds(i * 16, 16)] = scs[i]
        dig = dtot - 1
        sc_d1 = plsc.load_gather(scv, [jnp.minimum(dig + 1, 255)])
        sc_d1 = jnp.where(dig == 255, 0, sc_d1)
        need = need - sc_d1
        prefix = (prefix << 8) | dig

    thr = prefix

    def collect(j, cnt):
        key = keys[pl.ds(j * 16, 16)]
        m = key >= thr
        pos = cnt + plsc.cumsum(jnp.where(m, 1, 0)) - 1
        gidx = chunk * CH + j * 16 + lane
        plsc.store_scatter(candk, [pos], key, mask=m)
        plsc.store_scatter(candg, [pos], gidx, mask=m)
        cnt = cnt + jnp.max(plsc.all_reduce_population_count(m))
        return jnp.minimum(cnt, CAP - 16)
    lcnt = lax.fori_loop(0, NV, collect, jnp.zeros((), i32))

    lcnt16 = jnp.full((16,), 1, i32) * lcnt
    for j in range(8):
        cntv[pl.ds(j * 16, 16)] = lcnt16
    tid = bic * TPB + chunk
    pltpu.sync_copy(cntv, sh_cnt.at[pl.ds(tid * 128, 128)])
    pltpu.sync_copy(candk, sh_ck.at[pl.ds(tid * CAP, CAP)])
    pltpu.sync_copy(candg, sh_cg.at[pl.ds(tid * CAP, CAP)])
    plsc.subcore_barrier()

    pltpu.sync_copy(sh_cnt.at[pl.ds(bic * TPB * 128, TPB * 128)], cnts)
    pltpu.sync_copy(sh_ck.at[pl.ds(bic * TPB * CAP, TPB * CAP)], allk)
    pltpu.sync_copy(sh_cg.at[pl.ds(bic * TPB * CAP, TPB * CAP)], allg)

    def zero_out(j, carry):
        lout[pl.ds(j * 16, 16)] = zero16
        return carry
    lax.fori_loop(0, NOUT // 16, zero_out, 0)

    ngrp = (lcnt + 15) // 16

    def rank_group(g, carry):
        kc = candk[pl.ds(g * 16, 16)]
        gc = candg[pl.ds(g * 16, 16)]
        lanemask = (g * 16 + lane) < lcnt
        rank = zero16
        for r in range(TPB):
            cr = cnts[pl.ds(r * 128, 16)][0]
            roff = jnp.full((16,), r * CAP, i32)

            def inner(d, rk):
                kd = plsc.load_gather(allk, [roff + d])
                gd = plsc.load_gather(allg, [roff + d])
                gt = jnp.where(kd > kc, 1, 0)
                eq = jnp.where((kd == kc) & (gd < gc), 1, 0)
                return rk + gt + eq
            rank = lax.fori_loop(0, cr, inner, rank)
        outid = batch * N + gc
        plsc.store_scatter(lout, [rank], outid,
                           mask=lanemask & (rank < REL_THR))
        return carry
    lax.fori_loop(0, ngrp, rank_group, 0)

    pltpu.sync_copy(lout, sh_out.at[pl.ds(tid * NOUT, NOUT)])
    plsc.subcore_barrier()

    @pl.when(chunk == 0)
    def _():
        pltpu.sync_copy(sh_out.at[pl.ds(bic * TPB * NOUT, TPB * NOUT)], osum8)

        def merge(j, carry):
            acc = zero16
            for r in range(TPB):
                acc = acc + osum8[pl.ds(r * NOUT + j * 16, 16)]
            osum[pl.ds(j * 16, 16)] = acc
            return carry
        lax.fori_loop(0, NOUT // 16, merge, 0)
        pltpu.sync_copy(osum, out_hbm.at[pl.ds(batch * NOUT, NOUT)])


def _topk_sc(logits_flat):
    i32 = jnp.int32
    mesh = plsc.VectorSubcoreMesh(core_axis_name="c", subcore_axis_name="s",
                                  num_cores=2, num_subcores=16)
    f = pl.kernel(
        _topk_body,
        out_type=jax.ShapeDtypeStruct((B * NOUT,), i32),
        mesh=mesh,
        compiler_params=pltpu.CompilerParams(needs_layout_passes=False),
        scratch_types=[
            pltpu.VMEM((CH,), jnp.float32),
            pltpu.VMEM((CH,), i32),
            pltpu.VMEM((4096,), i32),
            pltpu.VMEM((256,), i32),
            pltpu.VMEM((TPB * 256,), i32),
            pltpu.VMEM((256,), i32),
            pltpu.VMEM((CAP,), i32),
            pltpu.VMEM((CAP,), i32),
            pltpu.VMEM((128,), i32),
            pltpu.VMEM((TPB * CAP,), i32),
            pltpu.VMEM((TPB * CAP,), i32),
            pltpu.VMEM((TPB * 128,), i32),
            pltpu.VMEM((NOUT,), i32),
            pltpu.VMEM((TPB * NOUT,), i32),
            pltpu.VMEM((NOUT,), i32),
            pltpu.VMEM_SHARED((2 * 4 * TPB * 256,), i32),
            pltpu.VMEM_SHARED((2 * TPB * CAP,), i32),
            pltpu.VMEM_SHARED((2 * TPB * CAP,), i32),
            pltpu.VMEM_SHARED((2 * TPB * 128,), i32),
            pltpu.VMEM_SHARED((2 * TPB * NOUT,), i32),
        ],
    )
    return f(logits_flat)


def kernel(feat_map, W_pre, b_pre, W_proj, b_proj):
    sel_logits = _logits_tc(feat_map, W_pre, b_pre, W_proj, b_proj)
    ids = _topk_sc(sel_logits.reshape(-1)).reshape(B, NOUT)
    sel_ids = ids[:, :REL_THR].reshape(-1)
    return sel_logits, sel_ids


# speedup vs baseline: 2.1093x; 1.1450x over previous
import jax
import jax.numpy as jnp
from jax import lax
from jax.experimental import pallas as pl
from jax.experimental.pallas import tpu as pltpu
from jax.experimental.pallas import tpu_sc as plsc

B, C, H, W = 4, 256, 64, 64
A = 9
REL_THR = 300
RB = 64
AP = 16

N = H * W * A
TPB = 8
CH = N // TPB
NV = CH // 16
CAP = 512
NOUT = 304


def _conv_body(x0, x1, x2, wt, wp, bpre, bproj, out):
    i = pl.program_id(1)
    xs = (x0, x1, x2)
    acc = jnp.zeros((RB * W, C), jnp.float32)
    for dy in range(3):
        start = (i * RB + dy) * W
        for dx in range(3):
            blk = xs[dx][0, pl.ds(start, RB * W), :]
            acc += jnp.dot(blk, wt[dy * 3 + dx],
                           preferred_element_type=jnp.float32)
    y = jnp.maximum(acc + bpre[0][None, :], 0.0)
    out[0] = jnp.dot(y, wp[...], preferred_element_type=jnp.float32) + bproj[0][None, :]


def _logits_tc(feat_map, W_pre, b_pre, W_proj, b_proj):
    x = jnp.transpose(feat_map, (0, 2, 3, 1))
    xp = jnp.pad(x, ((0, 0), (1, 1), (1, 1), (0, 0)))
    xs = [xp[:, :, dx:dx + W, :].reshape(B, (H + 2) * W, C) for dx in range(3)]
    wt = jnp.transpose(W_pre, (2, 3, 1, 0)).reshape(9, C, C)
    wp = jnp.pad(W_proj[:, :, 0, 0].T, ((0, 0), (0, AP - A)))
    bproj_p = jnp.pad(b_proj, (0, AP - A))

    grid = (B, H // RB)
    xspec = pl.BlockSpec((1, (H + 2) * W, C), lambda b, i: (b, 0, 0))
    out = pl.pallas_call(
        _conv_body,
        grid=grid,
        in_specs=[
            xspec, xspec, xspec,
            pl.BlockSpec((9, C, C), lambda b, i: (0, 0, 0)),
            pl.BlockSpec((C, AP), lambda b, i: (0, 0)),
            pl.BlockSpec((1, C), lambda b, i: (0, 0)),
            pl.BlockSpec((1, AP), lambda b, i: (0, 0)),
        ],
        out_specs=pl.BlockSpec((1, RB * W, AP), lambda b, i: (b, i, 0)),
        out_shape=jax.ShapeDtypeStruct((B, H * W, AP), jnp.float32),
    )(xs[0], xs[1], xs[2], wt, wp, b_pre[None, :], bproj_p[None, :])
    return out[:, :, :A].reshape(B, H * W * A)


def _topk_body(lg_hbm, out_hbm, vals, keys, h16, hist, hist8, scv, candk,
               candg, cntv, allk, allg, cnts, lout, osum8, osum, dmasem,
               sh_hist, sh_ck, sh_cg, sh_cnt, sh_out):
    i32 = jnp.int32
    c = lax.axis_index("c")
    s = lax.axis_index("s")
    bic = s // TPB
    chunk = s % TPB
    batch = c * 2 + bic
    lane = jnp.arange(16, dtype=i32)
    zero16 = jnp.zeros((16,), i32)
    one16 = jnp.ones((16,), i32)

    base = batch * N + chunk * CH
    cp = pltpu.async_copy(lg_hbm.at[pl.ds(base, CH)], vals, dmasem)

    def zero_h16(j, carry):
        h16[pl.ds(j * 16, 16)] = zero16
        return carry
    lax.fori_loop(0, 256, zero_h16, 0)
    cp.wait()

    lane_off = lane * 256

    def cvt(j, carry):
        v = vals[pl.ds(j * 16, 16)]
        bits = lax.bitcast_convert_type(v, i32)
        key = jnp.where(bits >= 0, bits, bits ^ jnp.int32(0x7FFFFFFF))
        keys[pl.ds(j * 16, 16)] = key
        d0 = (key >> 24) & 255
        plsc.addupdate_scatter(h16, [lane_off + d0], one16)
        return carry
    lax.fori_loop(0, NV, cvt, 0)

    need = jnp.full((16,), REL_THR, i32)
    prefix = jnp.zeros((16,), i32)

    for p in range(3):
        if p > 0:
            sh_hi = 32 - 8 * p
            pmask_bits = (1 << (8 * p)) - 1
            sh_d = 24 - 8 * p

            def scan(j, carry):
                key = keys[pl.ds(j * 16, 16)]
                pm = ((key >> sh_hi) & pmask_bits) == prefix
                d = (key >> sh_d) & 255
                plsc.addupdate_scatter(h16, [lane_off + d], one16, mask=pm)
                return carry
            lax.fori_loop(0, NV, scan, 0)

        def reduce_h(i, carry):
            acc = zero16
            for l in range(16):
                sl = pl.ds(l * 256 + i * 16, 16)
                acc = acc + h16[sl]
                h16[sl] = zero16
            hist[pl.ds(i * 16, 16)] = acc
            return carry
        lax.fori_loop(0, 16, reduce_h, 0)

        hrow = ((bic * 4 + p) * TPB + chunk) * 256
        pltpu.sync_copy(hist, sh_hist.at[pl.ds(hrow, 256)])
        plsc.subcore_barrier()
        pltpu.sync_copy(sh_hist.at[pl.ds((bic * 4 + p) * TPB * 256, TPB * 256)], hist8)

        hvecs = []
        for i in range(16):
            acc = zero16
            for r in range(TPB):
                acc = acc + hist8[pl.ds(r * 256 + i * 16, 16)]
            hvecs.append(acc)
        tail = jnp.zeros((), i32)
        scs = [None] * 16
        for i in range(15, -1, -1):
            ssum = jnp.flip(plsc.cumsum(jnp.flip(hvecs[i], 0)), 0)
            scs[i] = ssum + tail
            tail = tail + jnp.sum(hvecs[i])
        dtot = zero16
        for i in range(16):
            dtot = dtot + plsc.all_reduce_population_count(scs[i] >= need)
            scv[pl.ds(i * 16, 16)] = scs[i]
        dig = dtot - 1
        sc_d1 = plsc.load_gather(scv, [jnp.minimum(dig + 1, 255)])
        sc_d1 = jnp.where(dig == 255, 0, sc_d1)
        need = need - sc_d1
        prefix = (prefix << 8) | dig

    thr = prefix

    def collect(j, cnt):
        key = keys[pl.ds(j * 16, 16)]
        m = ((key >> 8) & 0xFFFFFF) >= thr
        pos = cnt + plsc.cumsum(jnp.where(m, 1, 0)) - 1
        gidx = chunk * CH + j * 16 + lane
        plsc.store_scatter(candk, [pos], key, mask=m)
        plsc.store_scatter(candg, [pos], gidx, mask=m)
        cnt = cnt + jnp.max(plsc.all_reduce_population_count(m))
        return jnp.minimum(cnt, CAP - 16)
    lcnt = lax.fori_loop(0, NV, collect, jnp.zeros((), i32))

    lcnt16 = jnp.full((16,), 1, i32) * lcnt
    for j in range(8):
        cntv[pl.ds(j * 16, 16)] = lcnt16
    tid = bic * TPB + chunk
    pltpu.sync_copy(cntv, sh_cnt.at[pl.ds(tid * 128, 128)])
    pltpu.sync_copy(candk, sh_ck.at[pl.ds(tid * CAP, CAP)])
    pltpu.sync_copy(candg, sh_cg.at[pl.ds(tid * CAP, CAP)])
    plsc.subcore_barrier()

    pltpu.sync_copy(sh_cnt.at[pl.ds(bic * TPB * 128, TPB * 128)], cnts)
    pltpu.sync_copy(sh_ck.at[pl.ds(bic * TPB * CAP, TPB * CAP)], allk)
    pltpu.sync_copy(sh_cg.at[pl.ds(bic * TPB * CAP, TPB * CAP)], allg)

    def zero_out(j, carry):
        lout[pl.ds(j * 16, 16)] = zero16
        return carry
    lax.fori_loop(0, NOUT // 16, zero_out, 0)

    ngrp = (lcnt + 15) // 16

    def rank_group(g, carry):
        kc = candk[pl.ds(g * 16, 16)]
        gc = candg[pl.ds(g * 16, 16)]
        lanemask = (g * 16 + lane) < lcnt
        rank = zero16
        for r in range(TPB):
            cr = cnts[pl.ds(r * 128, 16)][0]
            roff = jnp.full((16,), r * CAP, i32)

            def inner(d, rk):
                kd = plsc.load_gather(allk, [roff + d])
                gd = plsc.load_gather(allg, [roff + d])
                gt = jnp.where(kd > kc, 1, 0)
                eq = jnp.where((kd == kc) & (gd < gc), 1, 0)
                return rk + gt + eq
            rank = lax.fori_loop(0, cr, inner, rank)
        outid = batch * N + gc
        plsc.store_scatter(lout, [rank], outid,
                           mask=lanemask & (rank < REL_THR))
        return carry
    lax.fori_loop(0, ngrp, rank_group, 0)

    pltpu.sync_copy(lout, sh_out.at[pl.ds(tid * NOUT, NOUT)])
    plsc.subcore_barrier()

    @pl.when(chunk == 0)
    def _():
        pltpu.sync_copy(sh_out.at[pl.ds(bic * TPB * NOUT, TPB * NOUT)], osum8)

        def merge(j, carry):
            acc = zero16
            for r in range(TPB):
                acc = acc + osum8[pl.ds(r * NOUT + j * 16, 16)]
            osum[pl.ds(j * 16, 16)] = acc
            return carry
        lax.fori_loop(0, NOUT // 16, merge, 0)
        pltpu.sync_copy(osum, out_hbm.at[pl.ds(batch * NOUT, NOUT)])


def _topk_sc(logits_flat):
    i32 = jnp.int32
    mesh = plsc.VectorSubcoreMesh(core_axis_name="c", subcore_axis_name="s",
                                  num_cores=2, num_subcores=16)
    f = pl.kernel(
        _topk_body,
        out_type=jax.ShapeDtypeStruct((B * NOUT,), i32),
        mesh=mesh,
        compiler_params=pltpu.CompilerParams(needs_layout_passes=False),
        scratch_types=[
            pltpu.VMEM((CH,), jnp.float32),
            pltpu.VMEM((CH,), i32),
            pltpu.VMEM((4096,), i32),
            pltpu.VMEM((256,), i32),
            pltpu.VMEM((TPB * 256,), i32),
            pltpu.VMEM((256,), i32),
            pltpu.VMEM((CAP,), i32),
            pltpu.VMEM((CAP,), i32),
            pltpu.VMEM((128,), i32),
            pltpu.VMEM((TPB * CAP,), i32),
            pltpu.VMEM((TPB * CAP,), i32),
            pltpu.VMEM((TPB * 128,), i32),
            pltpu.VMEM((NOUT,), i32),
            pltpu.VMEM((TPB * NOUT,), i32),
            pltpu.VMEM((NOUT,), i32),
            pltpu.SemaphoreType.DMA,
            pltpu.VMEM_SHARED((2 * 4 * TPB * 256,), i32),
            pltpu.VMEM_SHARED((2 * TPB * CAP,), i32),
            pltpu.VMEM_SHARED((2 * TPB * CAP,), i32),
            pltpu.VMEM_SHARED((2 * TPB * 128,), i32),
            pltpu.VMEM_SHARED((2 * TPB * NOUT,), i32),
        ],
    )
    return f(logits_flat)


def kernel(feat_map, W_pre, b_pre, W_proj, b_proj):
    sel_logits = _logits_tc(feat_map, W_pre, b_pre, W_proj, b_proj)
    ids = _topk_sc(sel_logits.reshape(-1)).reshape(B, NOUT)
    sel_ids = ids[:, :REL_THR].reshape(-1)
    return sel_logits, sel_ids
